# Initial kernel scaffold; baseline (speedup 1.0000x reference)
import jax
import jax.numpy as jnp
from jax.experimental import pallas as pl


def kernel(x, y, memory, idx, params):
    raise NotImplementedError("write your pallas kernel here")


# speedup vs baseline: 2.0432x; 2.0432x over previous
import functools

import jax
import jax.numpy as jnp
from jax import lax
from jax.experimental import pallas as pl
from jax.experimental.pallas import tpu as pltpu
from jax.experimental.pallas import tpu_sc as plsc

NC = 2
NS = 16
L = 16
NW = NC * NS


def _splat(v, lane):
    return v.at[jnp.full((L,), lane, jnp.int32)].get(mode="promise_in_bounds")


def _build_sc(B, D, N, Kp1, interpret=False, nw=NW, nc=NC, ns=NS):
    BPW = B // nw
    CH = 128
    CPB = Kp1 // CH
    NCH = BPW * CPB
    DG = D // L
    assert B % nw == 0 and Kp1 % CH == 0 and D % L == 0 and BPW % 2 == 0

    mesh = plsc.VectorSubcoreMesh(core_axis_name="c", subcore_axis_name="s",
                                  num_cores=nc, num_subcores=ns)

    @functools.partial(
        pl.kernel,
        out_type=(jax.ShapeDtypeStruct((B, Kp1), jnp.float32),
                  jax.ShapeDtypeStruct((nw, L), jnp.float32)),
        mesh=mesh,
        interpret=interpret,
        compiler_params=None if interpret else pltpu.CompilerParams(
            needs_layout_passes=False),
        scratch_types=[
            pltpu.VMEM((NCH, CH), jnp.int32),
            pltpu.VMEM((BPW, D), jnp.float32),
            pltpu.VMEM((L,), jnp.float32),
            pltpu.VMEM((CH, D), jnp.float32),
            pltpu.VMEM((CH, D), jnp.float32),
            pltpu.VMEM((2, Kp1), jnp.float32),
            pltpu.VMEM((L,), jnp.float32),
            pltpu.SemaphoreType.DMA,
            pltpu.SemaphoreType.DMA,
            pltpu.SemaphoreType.DMA,
            pltpu.SemaphoreType.DMA,
        ],
    )
    def nce_sc(mem_hbm, x_hbm, idx_hbm, invt_hbm, e_hbm, sums_hbm,
               idx_v, x_v, invt_v, rows0_v, rows1_v, orow_v, acc_v,
               g0, g1, o0, o1):
        rows_bufs = (rows0_v, rows1_v)
        if interpret:
            w = jnp.int32(0)
        else:
            w = lax.axis_index("s") * nc + lax.axis_index("c")
        b0 = w * BPW
        pltpu.sync_copy(idx_hbm.at[pl.ds(w * NCH, NCH)], idx_v)
        pltpu.sync_copy(x_hbm.at[pl.ds(b0, BPW)], x_v)
        pltpu.sync_copy(invt_hbm, invt_v)
        acc_v[...] = jnp.zeros((L,), jnp.float32)
        invt = invt_v[...]
        iota = lax.iota(jnp.int32, L)
        gsems = (g0, g1)
        osems = (o0, o1)

        def fire(c, p):
            if interpret:
                src = mem_hbm.at[idx_v[c, :]]
            else:
                src = mem_hbm.at[idx_v.at[c]]
            pltpu.async_copy(src, rows_bufs[p], gsems[p])

        def wait_gather(p):
            pltpu.make_async_copy(mem_hbm.at[idx_v.at[0]], rows_bufs[p],
                                  gsems[p]).wait()

        def wait_orow(u):
            pltpu.make_async_copy(orow_v.at[u], e_hbm.at[b0], osems[u]).wait()

        def compute(b, q, p, u):
            rows = rows_bufs[p]

            def g_body(g, carry):
                ridx = iota + g * L
                accs = [jnp.zeros((L,), jnp.float32) for _ in range(4)]
                for dg in range(DG):
                    xv = x_v[b, pl.ds(dg * L, L)]
                    for lane in range(L):
                        xd = _splat(xv, lane)
                        cidx = jnp.full((L,), dg * L + lane, jnp.int32)
                        if interpret:
                            col = rows[...].at[ridx, cidx].get(
                                mode="promise_in_bounds")
                        else:
                            col = plsc.load_gather(rows, [ridx, cidx])
                        accs[lane % 4] = accs[lane % 4] + col * xd
                dot = (accs[0] + accs[1]) + (accs[2] + accs[3])
                e = jnp.exp(dot * invt)
                orow_v[u, pl.ds(q * CH + g * L, L)] = e
                acc_v[...] = acc_v[...] + e
                return carry

            lax.fori_loop(0, CH // L, g_body, 0)

        fire(0, 0)

        def b_body(bi, carry):
            for u in range(2):
                b = bi * 2 + u

                @pl.when(b >= 2)
                def _():
                    wait_orow(u)

                for q in range(CPB):
                    c = b * CPB + q
                    p = q & 1
                    if q < CPB - 1:
                        fire(c + 1, (q + 1) & 1)
                    else:
                        @pl.when(b < BPW - 1)
                        def _():
                            fire(c + 1, (q + 1) & 1)
                    wait_gather(p)
                    compute(b, q, p, u)
                pltpu.async_copy(orow_v.at[u], e_hbm.at[b0 + b], osems[u])
            return carry

        lax.fori_loop(0, BPW // 2, b_body, 0)
        wait_orow(0)
        wait_orow(1)
        pltpu.sync_copy(acc_v, sums_hbm.at[w])

    return nce_sc


def _norm_call(e, sums, params, N):
    B, Kp1 = e.shape

    def body(sums_ref, params_ref, e_ref, o_ref):
        s = jnp.sum(sums_ref[...])
        zval = params_ref[2]
        z = jnp.where(zval < 0.0, s * (float(N) / (B * Kp1)), zval)
        o_ref[...] = e_ref[...] / z

    return pl.pallas_call(
        body,
        out_shape=jax.ShapeDtypeStruct((B, Kp1), jnp.float32),
        in_specs=[
            pl.BlockSpec(memory_space=pltpu.VMEM),
            pl.BlockSpec(memory_space=pltpu.SMEM),
            pl.BlockSpec(memory_space=pltpu.VMEM),
        ],
        out_specs=pl.BlockSpec(memory_space=pltpu.VMEM),
    )(sums, params, e)


def kernel(x, y, memory, idx, params):
    B, D = x.shape
    N = memory.shape[0]
    Kp1 = idx.shape[1]
    idx = idx.at[:, 0].set(y.astype(idx.dtype))
    idx_r = idx.reshape(B * Kp1 // 128, 128).astype(jnp.int32)
    invt = jnp.full((L,), 1.0, jnp.float32) / params[1]
    mem_r = memory.astype(jnp.bfloat16).astype(jnp.float32)
    x_r = x.astype(jnp.bfloat16).astype(jnp.float32)
    e, sums = _build_sc(B, D, N, Kp1)(mem_r, x_r, idx_r, invt)
    return _norm_call(e, sums, params, N)


# speedup vs baseline: 8.9491x; 4.3800x over previous
import functools

import jax
import jax.numpy as jnp
from jax import lax
from jax.experimental import pallas as pl
from jax.experimental.pallas import tpu as pltpu
from jax.experimental.pallas import tpu_sc as plsc

NC = 2
NS = 16
L = 16
NW = NC * NS


def _splat(v, lane):
    return v.at[jnp.full((L,), lane, jnp.int32)].get(mode="promise_in_bounds")


def _build_sc(B, D, N, Kp1, interpret=False, nw=NW, nc=NC, ns=NS):
    BPW = B // nw
    CH = 128
    CPB = Kp1 // CH
    NCH = BPW * CPB
    DG = D // L
    assert B % nw == 0 and Kp1 % CH == 0 and D % L == 0 and BPW % 2 == 0

    mesh = plsc.VectorSubcoreMesh(core_axis_name="c", subcore_axis_name="s",
                                  num_cores=nc, num_subcores=ns)

    @functools.partial(
        pl.kernel,
        out_type=(jax.ShapeDtypeStruct((B, Kp1), jnp.float32),
                  jax.ShapeDtypeStruct((nw, L), jnp.float32)),
        mesh=mesh,
        interpret=interpret,
        compiler_params=None if interpret else pltpu.CompilerParams(
            needs_layout_passes=False),
        scratch_types=[
            pltpu.VMEM((NCH, CH), jnp.int32),
            pltpu.VMEM((BPW, D), jnp.float32),
            pltpu.VMEM((L,), jnp.float32),
            pltpu.VMEM((CH, D), jnp.float32),
            pltpu.VMEM((CH, D), jnp.float32),
            pltpu.VMEM((2, Kp1), jnp.float32),
            pltpu.VMEM((L,), jnp.float32),
            pltpu.SemaphoreType.DMA,
            pltpu.SemaphoreType.DMA,
            pltpu.SemaphoreType.DMA,
            pltpu.SemaphoreType.DMA,
        ],
    )
    def nce_sc(mem_hbm, x_hbm, idx_hbm, invt_hbm, e_hbm, sums_hbm,
               idx_v, x_v, invt_v, rows0_v, rows1_v, orow_v, acc_v,
               g0, g1, o0, o1):
        rows_bufs = (rows0_v, rows1_v)
        if interpret:
            w = jnp.int32(0)
        else:
            w = lax.axis_index("s") * nc + lax.axis_index("c")
        b0 = w * BPW
        pltpu.sync_copy(idx_hbm.at[pl.ds(w * NCH, NCH)], idx_v)
        pltpu.sync_copy(x_hbm.at[pl.ds(b0, BPW)], x_v)
        pltpu.sync_copy(invt_hbm, invt_v)
        acc_v[...] = jnp.zeros((L,), jnp.float32)
        invt = invt_v[...]
        iota = lax.iota(jnp.int32, L)
        gsems = (g0, g1)
        osems = (o0, o1)

        def fire(c, p):
            if interpret:
                src = mem_hbm.at[idx_v[c, :]]
            else:
                src = mem_hbm.at[idx_v.at[c]]
            pltpu.async_copy(src, rows_bufs[p], gsems[p])

        def wait_gather(p):
            pltpu.make_async_copy(mem_hbm.at[idx_v.at[0]], rows_bufs[p],
                                  gsems[p]).wait()

        def wait_orow(u):
            pltpu.make_async_copy(orow_v.at[u], e_hbm.at[b0], osems[u]).wait()

        def compute(b, q, p, u):
            rows = rows_bufs[p]
            xvs = [x_v[b, pl.ds(dg * L, L)] for dg in range(DG)]

            def g_body(g, carry):
                evec = jnp.zeros((L,), jnp.float32)
                for j in range(L):
                    r = g * L + j
                    prod = rows[r, pl.ds(0, L)] * xvs[0]
                    for dg in range(1, DG):
                        prod = prod + rows[r, pl.ds(dg * L, L)] * xvs[dg]
                    s = jnp.sum(prod)
                    evec = jnp.where(iota == j, s, evec)
                e = jnp.exp(evec * invt)
                orow_v[u, pl.ds(q * CH + g * L, L)] = e
                acc_v[...] = acc_v[...] + e
                return carry

            lax.fori_loop(0, CH // L, g_body, 0)

        fire(0, 0)

        def b_body(bi, carry):
            for u in range(2):
                b = bi * 2 + u

                @pl.when(b >= 2)
                def _():
                    wait_orow(u)

                for q in range(CPB):
                    c = b * CPB + q
                    p = q & 1
                    if q < CPB - 1:
                        fire(c + 1, (q + 1) & 1)
                    else:
                        @pl.when(b < BPW - 1)
                        def _():
                            fire(c + 1, (q + 1) & 1)
                    wait_gather(p)
                    compute(b, q, p, u)
                pltpu.async_copy(orow_v.at[u], e_hbm.at[b0 + b], osems[u])
            return carry

        lax.fori_loop(0, BPW // 2, b_body, 0)
        wait_orow(0)
        wait_orow(1)
        pltpu.sync_copy(acc_v, sums_hbm.at[w])

    return nce_sc


def _norm_call(e, sums, params, N):
    B, Kp1 = e.shape

    def body(sums_ref, params_ref, e_ref, o_ref):
        s = jnp.sum(sums_ref[...])
        zval = params_ref[2]
        z = jnp.where(zval < 0.0, s * (float(N) / (B * Kp1)), zval)
        o_ref[...] = e_ref[...] / z

    return pl.pallas_call(
        body,
        out_shape=jax.ShapeDtypeStruct((B, Kp1), jnp.float32),
        in_specs=[
            pl.BlockSpec(memory_space=pltpu.VMEM),
            pl.BlockSpec(memory_space=pltpu.SMEM),
            pl.BlockSpec(memory_space=pltpu.VMEM),
        ],
        out_specs=pl.BlockSpec(memory_space=pltpu.VMEM),
    )(sums, params, e)


def kernel(x, y, memory, idx, params):
    B, D = x.shape
    N = memory.shape[0]
    Kp1 = idx.shape[1]
    idx = idx.at[:, 0].set(y.astype(idx.dtype))
    idx_r = idx.reshape(B * Kp1 // 128, 128).astype(jnp.int32)
    invt = jnp.full((L,), 1.0, jnp.float32) / params[1]
    mem_r = memory.astype(jnp.bfloat16).astype(jnp.float32)
    x_r = x.astype(jnp.bfloat16).astype(jnp.float32)
    e, sums = _build_sc(B, D, N, Kp1)(mem_r, x_r, idx_r, invt)
    return _norm_call(e, sums, params, N)
